# baseline (speedup 1.0000x reference)
import numpy as np

import jax
import jax.numpy as jnp
from jax import lax
from jax.experimental import pallas as pl
from jax.experimental.pallas import tpu as pltpu
from jax.experimental.pallas import tpu_sc as plsc

B, T, F = 16384, 200, 9
ROWS = B * T
NW = 32
ROWS_PER_W = ROWS // NW
R = 6400
NCHUNK = ROWS_PER_W // R


def _body(x_hbm, o_st, o_kr, o_kc, o_ob, in_v, kr_v, ob_v, st_v, kc_v):
    wid = lax.axis_index("s") * 2 + lax.axis_index("c")

    lane = lax.iota(jnp.int32, 16)
    kr_idx = []
    for m in range(4):
        p = 16 * m + lane
        kr_idx.append(9 * (p >> 2) + 1 + (p & 3))
    ob_idx = []
    for m in range(3):
        p = 16 * m + lane
        q = (p * 21846) >> 16
        ob_idx.append(9 * q + 6 + (p - 3 * q))
    st_idx = 9 * lane
    kc_idx = 9 * lane + 5

    def chunk(i, carry):
        row0 = wid * ROWS_PER_W + i * R
        pltpu.sync_copy(x_hbm.at[pl.ds(row0 * F, R * F)], in_v)

        def deint(j, carry2):
            base = j * (16 * F)
            for m in range(4):
                v = plsc.load_gather(in_v, [base + kr_idx[m]])
                kr_v[pl.ds(j * 64 + 16 * m, 16)] = v
            for m in range(3):
                v = plsc.load_gather(in_v, [base + ob_idx[m]])
                ob_v[pl.ds(j * 48 + 16 * m, 16)] = v
            s = plsc.load_gather(in_v, [base + st_idx])
            st_v[pl.ds(j * 16, 16)] = s.astype(jnp.int32)
            c = plsc.load_gather(in_v, [base + kc_idx])
            kc_v[pl.ds(j * 16, 16)] = c.astype(jnp.int32)
            return carry2

        lax.fori_loop(0, R // 16, deint, 0, unroll=4)

        pltpu.sync_copy(kr_v, o_kr.at[pl.ds(row0 * 4, R * 4)])
        pltpu.sync_copy(ob_v, o_ob.at[pl.ds(row0 * 3, R * 3)])
        pltpu.sync_copy(st_v, o_st.at[pl.ds(row0, R)])
        pltpu.sync_copy(kc_v, o_kc.at[pl.ds(row0, R)])
        return carry

    lax.fori_loop(0, NCHUNK, chunk, 0)


@jax.jit
def _run(x_flat):
    mesh = plsc.VectorSubcoreMesh(core_axis_name="c", subcore_axis_name="s")
    f = pl.kernel(
        _body,
        out_type=[
            jax.ShapeDtypeStruct((ROWS,), jnp.int32),
            jax.ShapeDtypeStruct((ROWS * 4,), jnp.float32),
            jax.ShapeDtypeStruct((ROWS,), jnp.int32),
            jax.ShapeDtypeStruct((ROWS * 3,), jnp.float32),
        ],
        mesh=mesh,
        compiler_params=pltpu.CompilerParams(needs_layout_passes=False),
        scratch_types=[
            pltpu.VMEM((R * F,), jnp.float32),
            pltpu.VMEM((R * 4,), jnp.float32),
            pltpu.VMEM((R * 3,), jnp.float32),
            pltpu.VMEM((R,), jnp.int32),
            pltpu.VMEM((R,), jnp.int32),
        ],
    )
    return f(x_flat)


def kernel(inputs):
    st, kr, kc, ob = _run(inputs.reshape(ROWS * F))
    return (
        st.reshape(B, T, 1),
        kr.reshape(B, T, 4),
        kc.reshape(B, T, 1),
        ob.reshape(B, T, 3),
    )


# speedup vs baseline: 2.0277x; 2.0277x over previous
import functools

import jax
import jax.numpy as jnp
from jax import lax
from jax.experimental import pallas as pl
from jax.experimental.pallas import tpu as pltpu
from jax.experimental.pallas import tpu_sc as plsc

B, T, F = 16384, 200, 9
TT = T // 8
NW = 32
NG = 16


def _body(x_t, o_st, o_kr, o_kc, o_ob, in_v, cvt_v):
    wid = lax.axis_index("s") * 2 + lax.axis_index("c")

    lo = wid * 2 + jnp.minimum(wid, 11)
    hi = lo + 2 + (wid < 11).astype(jnp.int32)

    def ob_row(r, carry):
        p = (r >= 25).astype(jnp.int32) + (r >= 50).astype(jnp.int32)
        tt = r - 25 * p
        pltpu.sync_copy(
            x_t.at[6 + p, pl.ds(tt * 8, 8), :],
            o_ob.at[p, pl.ds(tt * 8, 8), :],
        )
        return carry

    lax.fori_loop(lo, hi, ob_row, 0)

    def kr_item(i, carry):
        idx = wid * 25 + i
        t = idx >> 2
        c = idx & 3
        pltpu.sync_copy(x_t.at[1 + c, t, :], o_kr.at[t, c, :])
        return carry

    lax.fori_loop(0, 25, kr_item, 0)

    def conv_plane(plane, out_ref):
        def group(g, carry):
            item = (wid & 15) * 25 + g
            tt = item >> 4
            bg = item & 15
            pltpu.sync_copy(
                x_t.at[plane, pl.ds(tt * 8, 8), pl.ds(bg * 1024, 1024)],
                in_v,
            )

            def cvt(j, carry2):
                for tr in range(8):
                    v = in_v[tr, pl.ds(j * 16, 16)]
                    cvt_v[tr, pl.ds(j * 16, 16)] = v.astype(jnp.int32)
                return carry2

            lax.fori_loop(0, 64, cvt, 0, unroll=4)
            for tr in range(8):
                pltpu.sync_copy(
                    cvt_v.at[tr],
                    out_ref.at[tt * 8 + tr, 0, pl.ds(bg * 1024, 1024)],
                )
            return carry

        lax.fori_loop(0, 25, group, 0)

    @pl.when(wid < 16)
    def _():
        conv_plane(0, o_st)

    @pl.when(wid >= 16)
    def _():
        conv_plane(5, o_kc)


@jax.jit
def _run(x_t):
    mesh = plsc.VectorSubcoreMesh(core_axis_name="c", subcore_axis_name="s")
    f = pl.kernel(
        _body,
        out_type=[
            jax.ShapeDtypeStruct((T, 1, B), jnp.int32),
            jax.ShapeDtypeStruct((T, 4, B), jnp.float32),
            jax.ShapeDtypeStruct((T, 1, B), jnp.int32),
            jax.ShapeDtypeStruct((3, T, B), jnp.float32),
        ],
        mesh=mesh,
        compiler_params=pltpu.CompilerParams(
            needs_layout_passes=False, use_tc_tiling_on_sc=True
        ),
        scratch_types=[
            pltpu.VMEM((8, 1024), jnp.float32),
            pltpu.VMEM((8, 1024), jnp.int32),
        ],
    )
    return f(x_t)


def kernel(inputs):
    x_t = jnp.transpose(inputs, (2, 1, 0))
    st, kr, kc, ob = _run(x_t)
    return (
        jnp.transpose(st, (2, 0, 1)),
        jnp.transpose(kr, (2, 0, 1)),
        jnp.transpose(kc, (2, 0, 1)),
        jnp.transpose(ob, (2, 1, 0)),
    )


# speedup vs baseline: 37.4999x; 18.4941x over previous
import functools

import jax
import jax.numpy as jnp
from jax import lax
from jax.experimental import pallas as pl
from jax.experimental.pallas import tpu as pltpu
from jax.experimental.pallas import tpu_sc as plsc

B, T, F = 16384, 200, 9
TT = T // 8
NW = 32


def _staged_ring(lo, hi, src_of, dst_of, bufs, sems_i, sems_o):
    @pl.when(hi > lo)
    def _():
        pltpu.async_copy(src_of(lo), bufs[0], sems_i[0])

        def step(i, carry):
            for b in range(2):
                @pl.when(((i - lo) & 1) == b)
                def _(b=b):
                    pltpu.make_async_copy(src_of(i), bufs[b],
                                          sems_i[b]).wait()

                    @pl.when(i > lo)
                    def _():
                        pltpu.make_async_copy(bufs[1 - b], dst_of(i - 1),
                                              sems_o[1 - b]).wait()

                    @pl.when(i + 1 < hi)
                    def _():
                        pltpu.async_copy(src_of(i + 1), bufs[1 - b],
                                         sems_i[1 - b])

                    pltpu.async_copy(bufs[b], dst_of(i), sems_o[b])
            return carry

        lax.fori_loop(lo, hi, step, 0)
        for b in range(2):
            @pl.when(((hi - 1 - lo) & 1) == b)
            def _(b=b):
                pltpu.make_async_copy(bufs[b], dst_of(hi - 1),
                                      sems_o[b]).wait()


def _body(x_t, o_st, o_kr, o_kc, o_ob,
          ob0, ob1, kr0, kr1, in_v0, in_v1, cvt_v0, cvt_v1,
          s_i0, s_i1, s_o0, s_o1, c_i0, c_i1, c_o0, c_o1):
    wid = lax.axis_index("s") * 2 + lax.axis_index("c")

    def ob_src(i):
        p = (i >= 200).astype(jnp.int32) + (i >= 400).astype(jnp.int32)
        r = i - 200 * p
        return x_t.at[6 + p, pl.ds((r >> 3) * 8, 8),
                      pl.ds((r & 7) * 2048, 2048)]

    def ob_dst(i):
        p = (i >= 200).astype(jnp.int32) + (i >= 400).astype(jnp.int32)
        r = i - 200 * p
        return o_ob.at[p, pl.ds((r >> 3) * 8, 8),
                       pl.ds((r & 7) * 2048, 2048)]

    _staged_ring((wid * 75) >> 2, ((wid + 1) * 75) >> 2, ob_src, ob_dst,
                 (ob0, ob1), (s_i0, s_i1), (s_o0, s_o1))

    def kr_src(i):
        return x_t.at[pl.ds(1, 4), i >> 2, pl.ds((i & 3) * 4096, 4096)]

    def kr_dst(i):
        return o_kr.at[i >> 2, :, pl.ds((i & 3) * 4096, 4096)]

    _staged_ring(wid * 25, (wid + 1) * 25, kr_src, kr_dst,
                 (kr0, kr1), (c_i0, c_i1), (c_o0, c_o1))

    def conv_plane(plane, out_ref):
        in_bufs = (in_v0, in_v1)
        cvt_bufs = (cvt_v0, cvt_v1)
        in_sems = (s_i0, s_i1)
        out_sems = (s_o0, s_o1)

        def src_of(g):
            item = (wid & 15) * 25 + g
            return x_t.at[plane, pl.ds((item >> 4) * 8, 8),
                          pl.ds((item & 15) * 1024, 1024)]

        def dst_of(g):
            item = (wid & 15) * 25 + g
            return out_ref.at[pl.ds((item >> 4) * 8, 8), 0,
                              pl.ds((item & 15) * 1024, 1024)]

        pltpu.async_copy(src_of(0), in_bufs[0], in_sems[0])

        def step(g, carry):
            for b in range(2):
                @pl.when((g & 1) == b)
                def _(b=b):
                    pltpu.make_async_copy(src_of(g), in_bufs[b],
                                          in_sems[b]).wait()

                    @pl.when(g < 24)
                    def _():
                        pltpu.async_copy(src_of(g + 1), in_bufs[1 - b],
                                         in_sems[1 - b])

                    @pl.when(g >= 2)
                    def _():
                        pltpu.make_async_copy(cvt_bufs[b], dst_of(g - 2),
                                              out_sems[b]).wait()

                    def cvt(j, carry2):
                        for tr in range(8):
                            v = in_bufs[b][tr, pl.ds(j * 16, 16)]
                            cvt_bufs[b][tr, pl.ds(j * 16, 16)] = (
                                v.astype(jnp.int32))
                        return carry2

                    lax.fori_loop(0, 64, cvt, 0, unroll=4)
                    pltpu.async_copy(cvt_bufs[b], dst_of(g), out_sems[b])
            return carry

        lax.fori_loop(0, 25, step, 0)
        pltpu.make_async_copy(cvt_bufs[1], dst_of(23), out_sems[1]).wait()
        pltpu.make_async_copy(cvt_bufs[0], dst_of(24), out_sems[0]).wait()

    @pl.when(wid < 16)
    def _():
        conv_plane(0, o_st)

    @pl.when(wid >= 16)
    def _():
        conv_plane(5, o_kc)


@jax.jit
def _run(x_t):
    mesh = plsc.VectorSubcoreMesh(core_axis_name="c", subcore_axis_name="s")
    f = pl.kernel(
        _body,
        out_type=[
            jax.ShapeDtypeStruct((T, 1, B), jnp.int32),
            jax.ShapeDtypeStruct((T, 4, B), jnp.float32),
            jax.ShapeDtypeStruct((T, 1, B), jnp.int32),
            jax.ShapeDtypeStruct((3, T, B), jnp.float32),
        ],
        mesh=mesh,
        compiler_params=pltpu.CompilerParams(
            needs_layout_passes=False, use_tc_tiling_on_sc=True
        ),
        scratch_types=[
            pltpu.VMEM((8, 2048), jnp.float32),
            pltpu.VMEM((8, 2048), jnp.float32),
            pltpu.VMEM((4, 4096), jnp.float32),
            pltpu.VMEM((4, 4096), jnp.float32),
            pltpu.VMEM((8, 1024), jnp.float32),
            pltpu.VMEM((8, 1024), jnp.float32),
            pltpu.VMEM((8, 1024), jnp.int32),
            pltpu.VMEM((8, 1024), jnp.int32),
            pltpu.SemaphoreType.DMA,
            pltpu.SemaphoreType.DMA,
            pltpu.SemaphoreType.DMA,
            pltpu.SemaphoreType.DMA,
            pltpu.SemaphoreType.DMA,
            pltpu.SemaphoreType.DMA,
            pltpu.SemaphoreType.DMA,
            pltpu.SemaphoreType.DMA,
        ],
    )
    return f(x_t)


def kernel(inputs):
    x_t = jnp.transpose(inputs, (2, 1, 0))
    st, kr, kc, ob = _run(x_t)
    return (
        jnp.transpose(st, (2, 0, 1)),
        jnp.transpose(kr, (2, 0, 1)),
        jnp.transpose(kc, (2, 0, 1)),
        jnp.transpose(ob, (2, 1, 0)),
    )
